# baseline (speedup 1.0000x reference)
import jax
import jax.numpy as jnp
from jax.experimental import pallas as pl
from jax.experimental.pallas import tpu as pltpu

MAXLEN = 1024


def _interval_count_mask(r, starts_ref, ends_ref, nb):
    cnt = jnp.zeros(r.shape, jnp.int32)
    for b in range(nb):
        s = starts_ref[b]
        e = ends_ref[b]
        cnt = cnt + (r >= s).astype(jnp.int32) - (r >= e).astype(jnp.int32)
    return (cnt > 0).astype(jnp.float32)


def _node_mask_body(starts_ref, ends_ref, out_ref):
    g, lanes = out_ref.shape
    i = pl.program_id(0)
    r = (i * g + jax.lax.broadcasted_iota(jnp.int32, (g, lanes), 0)) * lanes \
        + jax.lax.broadcasted_iota(jnp.int32, (g, lanes), 1)
    out_ref[...] = _interval_count_mask(r, starts_ref, ends_ref,
                                        starts_ref.shape[0])


def _atom_mask_body(starts_ref, ends_ref, a2r_ref, out_ref):
    r = a2r_ref[...]
    out_ref[...] = _interval_count_mask(r, starts_ref, ends_ref,
                                        starts_ref.shape[0])


def _build_masks(starts, ends, atom2residue, L, A):
    LANES = 128
    GL, GA = 256, 256
    node_mask = pl.pallas_call(
        _node_mask_body,
        grid=(L // (GL * LANES),),
        in_specs=[
            pl.BlockSpec(memory_space=pltpu.SMEM),
            pl.BlockSpec(memory_space=pltpu.SMEM),
        ],
        out_specs=pl.BlockSpec((GL, LANES), lambda i: (i, 0)),
        out_shape=jax.ShapeDtypeStruct((L // LANES, LANES), jnp.float32),
    )(starts, ends)
    a2r2d = atom2residue.reshape(A // LANES, LANES)
    atom_mask = pl.pallas_call(
        _atom_mask_body,
        grid=(A // (GA * LANES),),
        in_specs=[
            pl.BlockSpec(memory_space=pltpu.SMEM),
            pl.BlockSpec(memory_space=pltpu.SMEM),
            pl.BlockSpec((GA, LANES), lambda i: (i, 0)),
        ],
        out_specs=pl.BlockSpec((GA, LANES), lambda i: (i, 0)),
        out_shape=jax.ShapeDtypeStruct((A // LANES, LANES), jnp.float32),
    )(starts, ends, a2r2d)
    return node_mask.reshape(L, 1), atom_mask.reshape(A, 1)


NBUF = 4


def _skip_mul_body(starts_ref, ends_ref, rmin_ref, rmax_ref,
                   mask_ref, feat_hbm, out_ref, scratch, sems):
    i = pl.program_id(0)
    n = pl.num_programs(0)
    nb = starts_ref.shape[0]
    br = out_ref.shape[0]

    def nonzero(j):
        acc = None
        for b in range(nb):
            hit = (starts_ref[b] <= rmax_ref[j]) & (ends_ref[b] > rmin_ref[j])
            acc = hit if acc is None else (acc | hit)
        return acc

    def start_dma(j, slot):
        pltpu.make_async_copy(
            feat_hbm.at[pl.ds(j * br, br), :], scratch.at[slot],
            sems.at[slot]).start()

    @pl.when(i == 0)
    def _():
        for j in range(NBUF - 1):
            @pl.when(jnp.logical_and(j < n, nonzero(jnp.minimum(j, n - 1))))
            def _():
                start_dma(j, j % NBUF)

    nxt = jnp.minimum(i + NBUF - 1, n - 1)

    @pl.when(jnp.logical_and(i + NBUF - 1 < n, nonzero(nxt)))
    def _():
        start_dma(nxt, jax.lax.rem(i + NBUF - 1, NBUF))

    live = nonzero(i)

    @pl.when(live)
    def _():
        slot = jax.lax.rem(i, NBUF)
        pltpu.make_async_copy(
            feat_hbm.at[pl.ds(i * br, br), :], scratch.at[slot],
            sems.at[slot]).wait()
        out_ref[...] = scratch[slot] * mask_ref[...]

    @pl.when(jnp.logical_not(live))
    def _():
        out_ref[...] = jnp.zeros_like(out_ref)


def _masked_mul_skip(feat, mask_col, starts, ends, rmin, rmax, block_rows):
    n, d = feat.shape
    grid = n // block_rows
    return pl.pallas_call(
        _skip_mul_body,
        grid_spec=pltpu.PrefetchScalarGridSpec(
            num_scalar_prefetch=4,
            grid=(grid,),
            in_specs=[
                pl.BlockSpec((block_rows, 1), lambda i, *_: (i, 0)),
                pl.BlockSpec(memory_space=pltpu.MemorySpace.HBM),
            ],
            out_specs=pl.BlockSpec((block_rows, d), lambda i, *_: (i, 0)),
            scratch_shapes=[
                pltpu.VMEM((NBUF, block_rows, d), feat.dtype),
                pltpu.SemaphoreType.DMA((NBUF,)),
            ],
        ),
        out_shape=jax.ShapeDtypeStruct((n, d), feat.dtype),
        compiler_params=pltpu.CompilerParams(
            dimension_semantics=("arbitrary",)),
    )(starts, ends, rmin, rmax, mask_col, feat)


def kernel(residue_feat, atom_feat, rand_u, num_residues, atom2residue):
    L, D = residue_feat.shape
    A = atom_feat.shape[0]
    num_cum = jnp.cumsum(num_residues)
    starts_local = (rand_u * jnp.clip(num_residues - MAXLEN, 0, None)
                    .astype(jnp.float32)).astype(jnp.int32)
    ends_local = jnp.minimum(starts_local + MAXLEN, num_residues)
    offset = num_cum - num_residues
    starts = starts_local + offset
    ends = ends_local + offset

    node_mask, atom_mask = _build_masks(starts, ends, atom2residue, L, A)

    BR, BA = 1024, 1024
    res_rmin = jnp.arange(0, L, BR, dtype=jnp.int32)
    res_rmax = res_rmin + (BR - 1)
    atom_rmin = atom2residue[0::BA]
    atom_rmax = atom2residue[BA - 1::BA]

    out_residue = _masked_mul_skip(residue_feat, node_mask, starts, ends,
                                   res_rmin, res_rmax, BR)
    out_atom = _masked_mul_skip(atom_feat, atom_mask, starts, ends,
                                atom_rmin, atom_rmax, BA)
    return out_residue, out_atom


# speedup vs baseline: 1.1942x; 1.1942x over previous
import jax
import jax.numpy as jnp
from jax.experimental import pallas as pl
from jax.experimental.pallas import tpu as pltpu

MAXLEN = 1024


def _interval_count_mask(r, starts_ref, ends_ref, nb):
    cnt = jnp.zeros(r.shape, jnp.int32)
    for b in range(nb):
        s = starts_ref[b]
        e = ends_ref[b]
        cnt = cnt + (r >= s).astype(jnp.int32) - (r >= e).astype(jnp.int32)
    return (cnt > 0).astype(jnp.float32)


def _node_mask_body(starts_ref, ends_ref, out_ref):
    g, lanes = out_ref.shape
    i = pl.program_id(0)
    r = (i * g + jax.lax.broadcasted_iota(jnp.int32, (g, lanes), 0)) * lanes \
        + jax.lax.broadcasted_iota(jnp.int32, (g, lanes), 1)
    out_ref[...] = _interval_count_mask(r, starts_ref, ends_ref,
                                        starts_ref.shape[0])


def _atom_mask_body(starts_ref, ends_ref, a2r_ref, out_ref):
    r = a2r_ref[...]
    out_ref[...] = _interval_count_mask(r, starts_ref, ends_ref,
                                        starts_ref.shape[0])


def _build_masks(starts, ends, atom2residue, L, A):
    LANES = 128
    GL, GA = 256, 256
    node_mask = pl.pallas_call(
        _node_mask_body,
        grid=(L // (GL * LANES),),
        in_specs=[
            pl.BlockSpec(memory_space=pltpu.SMEM),
            pl.BlockSpec(memory_space=pltpu.SMEM),
        ],
        out_specs=pl.BlockSpec((GL, LANES), lambda i: (i, 0)),
        out_shape=jax.ShapeDtypeStruct((L // LANES, LANES), jnp.float32),
    )(starts, ends)
    a2r2d = atom2residue.reshape(A // LANES, LANES)
    atom_mask = pl.pallas_call(
        _atom_mask_body,
        grid=(A // (GA * LANES),),
        in_specs=[
            pl.BlockSpec(memory_space=pltpu.SMEM),
            pl.BlockSpec(memory_space=pltpu.SMEM),
            pl.BlockSpec((GA, LANES), lambda i: (i, 0)),
        ],
        out_specs=pl.BlockSpec((GA, LANES), lambda i: (i, 0)),
        out_shape=jax.ShapeDtypeStruct((A // LANES, LANES), jnp.float32),
    )(starts, ends, a2r2d)
    return node_mask.reshape(L, 1), atom_mask.reshape(A, 1)


NBUF = 4


def _skip_mul_body(starts_ref, ends_ref, rmin_ref, rmax_ref,
                   mask_ref, feat_hbm, out_ref, scratch, sems):
    i = pl.program_id(0)
    n = pl.num_programs(0)
    nb = starts_ref.shape[0]
    br = out_ref.shape[0]

    def nonzero(j):
        acc = None
        for b in range(nb):
            hit = (starts_ref[b] <= rmax_ref[j]) & (ends_ref[b] > rmin_ref[j])
            acc = hit if acc is None else (acc | hit)
        return acc

    def start_dma(j, slot):
        pltpu.make_async_copy(
            feat_hbm.at[pl.ds(j * br, br), :], scratch.at[slot],
            sems.at[slot]).start()

    @pl.when(i == 0)
    def _():
        for j in range(NBUF - 1):
            @pl.when(jnp.logical_and(j < n, nonzero(jnp.minimum(j, n - 1))))
            def _():
                start_dma(j, j % NBUF)

    nxt = jnp.minimum(i + NBUF - 1, n - 1)

    @pl.when(jnp.logical_and(i + NBUF - 1 < n, nonzero(nxt)))
    def _():
        start_dma(nxt, jax.lax.rem(i + NBUF - 1, NBUF))

    live = nonzero(i)

    @pl.when(live)
    def _():
        slot = jax.lax.rem(i, NBUF)
        pltpu.make_async_copy(
            feat_hbm.at[pl.ds(i * br, br), :], scratch.at[slot],
            sems.at[slot]).wait()
        out_ref[...] = scratch[slot] * mask_ref[...]

    @pl.when(jnp.logical_not(live))
    def _():
        out_ref[...] = jnp.zeros_like(out_ref)


def _masked_mul_skip(feat, mask_col, starts, ends, rmin, rmax, block_rows):
    n, d = feat.shape
    grid = n // block_rows
    return pl.pallas_call(
        _skip_mul_body,
        grid_spec=pltpu.PrefetchScalarGridSpec(
            num_scalar_prefetch=4,
            grid=(grid,),
            in_specs=[
                pl.BlockSpec((block_rows, 1), lambda i, *_: (i, 0)),
                pl.BlockSpec(memory_space=pltpu.MemorySpace.HBM),
            ],
            out_specs=pl.BlockSpec((block_rows, d), lambda i, *_: (i, 0)),
            scratch_shapes=[
                pltpu.VMEM((NBUF, block_rows, d), feat.dtype),
                pltpu.SemaphoreType.DMA((NBUF,)),
            ],
        ),
        out_shape=jax.ShapeDtypeStruct((n, d), feat.dtype),
        compiler_params=pltpu.CompilerParams(
            dimension_semantics=("arbitrary",)),
    )(starts, ends, rmin, rmax, mask_col, feat)


def kernel(residue_feat, atom_feat, rand_u, num_residues, atom2residue):
    L, D = residue_feat.shape
    A = atom_feat.shape[0]
    num_cum = jnp.cumsum(num_residues)
    starts_local = (rand_u * jnp.clip(num_residues - MAXLEN, 0, None)
                    .astype(jnp.float32)).astype(jnp.int32)
    ends_local = jnp.minimum(starts_local + MAXLEN, num_residues)
    offset = num_cum - num_residues
    starts = starts_local + offset
    ends = ends_local + offset

    node_mask, atom_mask = _build_masks(starts, ends, atom2residue, L, A)

    BR, BA = 2048, 2048
    res_rmin = jnp.arange(0, L, BR, dtype=jnp.int32)
    res_rmax = res_rmin + (BR - 1)
    atom_rmin = atom2residue[0::BA]
    atom_rmax = atom2residue[BA - 1::BA]

    out_residue = _masked_mul_skip(residue_feat, node_mask, starts, ends,
                                   res_rmin, res_rmax, BR)
    out_atom = _masked_mul_skip(atom_feat, atom_mask, starts, ends,
                                atom_rmin, atom_rmax, BA)
    return out_residue, out_atom
